# baseline (speedup 1.0000x reference)
import functools

import jax
import jax.numpy as jnp
from jax import lax
from jax.experimental import pallas as pl
from jax.experimental.pallas import tpu as pltpu
from jax.experimental.pallas import tpu_sc as plsc

VOCAB = 1000000
EMB = 32
C = 6
B = 16384
NC = 2
NS = 16
NW = NC * NS
BPW = B // NW
CPW = BPW * C
WB = 16
WC = WB * C
WAVES = BPW // WB

_mesh = plsc.VectorSubcoreMesh(core_axis_name="c", subcore_axis_name="s")


@functools.partial(
    pl.kernel,
    mesh=_mesh,
    compiler_params=pltpu.CompilerParams(
        needs_layout_passes=False, use_tc_tiling_on_sc=False),
    out_type=jax.ShapeDtypeStruct((B * C,), jnp.float32),
    scratch_types=[
        pltpu.VMEM((BPW,), jnp.int32),
        pltpu.VMEM((CPW,), jnp.int32),
        pltpu.VMEM((EMB, WB * 16), jnp.float32),
        pltpu.VMEM((EMB, WB * 16), jnp.float32),
        pltpu.VMEM((EMB, WC * 16), jnp.float32),
        pltpu.VMEM((EMB, WC * 16), jnp.float32),
        pltpu.VMEM((CPW,), jnp.float32),
        pltpu.SemaphoreType.DMA,
        pltpu.SemaphoreType.DMA,
    ],
)
def _w2v(tgt_hbm, ctx_hbm, ttab_hbm, ctab_hbm, out_hbm,
         tidx_v, cidx_v, tb0, tb1, cb0, cb1, out_v, sem0, sem1):
    wid = lax.axis_index("s") * NC + lax.axis_index("c")
    tbase = wid * BPW
    cbase = wid * CPW

    pltpu.sync_copy(tgt_hbm.at[pl.ds(tbase, BPW)], tidx_v)
    pltpu.sync_copy(ctx_hbm.at[pl.ds(cbase, CPW)], cidx_v)

    iota16 = lax.iota(jnp.int32, 16)

    def fire(w, tb, cb, sem):
        tvec = tidx_v[pl.ds(w * WB, 16)] & -16
        for j in range(WB):
            a = pl.multiple_of(tvec[j], 16)
            pltpu.async_copy(ttab_hbm.at[:, pl.ds(a, 16)],
                             tb.at[:, pl.ds(j * 16, 16)], sem)
        for g in range(C):
            cvec = cidx_v[pl.ds(w * WC + g * 16, 16)] & -16
            for j in range(16):
                jj = g * 16 + j
                a = pl.multiple_of(cvec[j], 16)
                pltpu.async_copy(ctab_hbm.at[:, pl.ds(a, 16)],
                                 cb.at[:, pl.ds(jj * 16, 16)], sem)

    def drain(tb, cb, sem):
        def dt(j, carry):
            pltpu.make_async_copy(ttab_hbm.at[:, pl.ds(0, 16)],
                                  tb.at[:, pl.ds(0, 16)], sem).wait()
            return carry

        def dc(j, carry):
            pltpu.make_async_copy(ctab_hbm.at[:, pl.ds(0, 16)],
                                  cb.at[:, pl.ds(0, 16)], sem).wait()
            return carry

        lax.fori_loop(0, WB, dt, 0)
        lax.fori_loop(0, WC, dc, 0)

    def compute(w, tb, cb):
        toff = tidx_v[pl.ds(w * WB, 16)] & 15
        tcol = iota16 * 16 + toff
        r_vecs = [iota16 * C + c for c in range(C)]
        ccols = [
            (plsc.load_gather(cidx_v, [w * WC + r]) & 15) + r * 16
            for r in r_vecs
        ]
        acc = [jnp.zeros((16,), jnp.float32) for _ in range(C)]
        fe = jnp.zeros((16,), jnp.int32)
        ones = jnp.ones((16,), jnp.int32)
        for e in range(EMB):
            tv = plsc.load_gather(tb, [fe, tcol])
            for c in range(C):
                cv = plsc.load_gather(cb, [fe, ccols[c]])
                acc[c] = acc[c] + tv * cv
            fe = fe + ones
        for c in range(C):
            plsc.store_scatter(out_v, [w * WC + r_vecs[c]], acc[c])

    fire(0, tb0, cb0, sem0)
    fire(1, tb1, cb1, sem1)

    def body(k, carry):
        w0 = 2 * k
        drain(tb0, cb0, sem0)
        compute(w0, tb0, cb0)

        @pl.when(w0 + 2 < WAVES)
        def _():
            fire(w0 + 2, tb0, cb0, sem0)

        w1 = 2 * k + 1
        drain(tb1, cb1, sem1)
        compute(w1, tb1, cb1)

        @pl.when(w1 + 2 < WAVES)
        def _():
            fire(w1 + 2, tb1, cb1, sem1)

        return carry

    lax.fori_loop(0, WAVES // 2, body, 0)

    pltpu.sync_copy(out_v, out_hbm.at[pl.ds(cbase, CPW)])


def kernel(tgt, ctx, target_table, context_table):
    out = _w2v(tgt.reshape(-1), ctx.reshape(-1),
               target_table.T, context_table.T)
    return out.reshape(B, C)


# speedup vs baseline: 5.7707x; 5.7707x over previous
import functools

import jax
import jax.numpy as jnp
from jax import lax
from jax.experimental import pallas as pl
from jax.experimental.pallas import tpu as pltpu
from jax.experimental.pallas import tpu_sc as plsc

VOCAB = 1000000
EMB = 32
C = 6
B = 16384
NC = 2
NS = 16
NW = NC * NS
BPW = B // NW
CPW = BPW * C
CHUNK = 128

_mesh = plsc.VectorSubcoreMesh(core_axis_name="c", subcore_axis_name="s")


@functools.partial(
    pl.kernel,
    mesh=_mesh,
    compiler_params=pltpu.CompilerParams(
        needs_layout_passes=False, use_tc_tiling_on_sc=False),
    out_type=jax.ShapeDtypeStruct((B * C,), jnp.float32),
    scratch_types=[
        pltpu.VMEM((BPW,), jnp.int32),
        pltpu.VMEM((CPW,), jnp.int32),
        pltpu.VMEM((BPW, EMB), jnp.float32),
        pltpu.VMEM((CPW, EMB), jnp.float32),
        pltpu.VMEM((CPW,), jnp.float32),
        pltpu.SemaphoreType.DMA,
    ],
)
def _w2v(tgt_hbm, ctx_hbm, ttab_hbm, ctab_hbm, out_hbm,
         tidx_v, cidx_v, trow_v, crow_v, out_v, sem):
    wid = lax.axis_index("s") * NC + lax.axis_index("c")
    tbase = wid * BPW
    cbase = wid * CPW

    pltpu.sync_copy(tgt_hbm.at[pl.ds(tbase, BPW)], tidx_v)
    pltpu.sync_copy(ctx_hbm.at[pl.ds(cbase, CPW)], cidx_v)

    handles = []
    for j in range(BPW // CHUNK):
        handles.append(pltpu.async_copy(
            ttab_hbm.at[tidx_v.at[pl.ds(j * CHUNK, CHUNK)]],
            trow_v.at[pl.ds(j * CHUNK, CHUNK)], sem))
    for j in range(CPW // CHUNK):
        handles.append(pltpu.async_copy(
            ctab_hbm.at[cidx_v.at[pl.ds(j * CHUNK, CHUNK)]],
            crow_v.at[pl.ds(j * CHUNK, CHUNK)], sem))
    for h in handles:
        h.wait()

    iota16 = lax.iota(jnp.int32, 16)

    def body(blk, carry):
        b_vec = blk * 16 + iota16
        r_vecs = [b_vec * C + c for c in range(C)]
        acc = [jnp.zeros((16,), jnp.float32) for _ in range(C)]
        for e in range(EMB):
            col = jnp.full((16,), e, jnp.int32)
            tv = plsc.load_gather(trow_v, [b_vec, col])
            for c in range(C):
                cv = plsc.load_gather(crow_v, [r_vecs[c], col])
                acc[c] = acc[c] + tv * cv
        for c in range(C):
            plsc.store_scatter(out_v, [r_vecs[c]], acc[c])
        return carry

    lax.fori_loop(0, BPW // 16, body, 0)

    pltpu.sync_copy(out_v, out_hbm.at[pl.ds(cbase, CPW)])


def kernel(tgt, ctx, target_table, context_table):
    out = _w2v(tgt.reshape(-1), ctx.reshape(-1), target_table, context_table)
    return out.reshape(B, C)
